# Initial kernel scaffold; baseline (speedup 1.0000x reference)
import jax
import jax.numpy as jnp
from jax.experimental import pallas as pl


def kernel(x, rel_base, rel_weight, W1_msg, W1_root, b1, W2_msg, W2_root, b2, edge_index, edge_type, target_node):
    raise NotImplementedError("write your pallas kernel here")


# speedup vs baseline: 4.1438x; 4.1438x over previous
import jax
import jax.numpy as jnp
from jax import lax
from jax.experimental import pallas as pl
from jax.experimental.pallas import tpu as pltpu
from jax.experimental.pallas import tpu_sc as plsc

_N = 10000
_E = 640000
_D = 100
_H = 50
_C = 40
_T = 100
_DP1 = 128
_DP2 = 128

_NC = 2
_NS = 16
_NW = _NC * _NS
_EPW = _E // _NW
_CH = 80
_NCHUNK = _EPW // _CH
_NP = 10240
_RPS = _NP // _NS
_ZR = 128


def _make_edge_pass(dp):
  nj = dp // 16
  mesh = plsc.VectorSubcoreMesh(core_axis_name="c", subcore_axis_name="s")

  def body(table, rel, src, dst, et, out, acc, rel_v, rows_v, src_v, dst_v,
           et_v, z_v, sem):
    cid = lax.axis_index("c")
    sid = lax.axis_index("s")
    wid = sid * _NC + cid

    pltpu.sync_copy(rel, rel_v)

    def zrow(r, carry):
      for j in range(nj):
        z_v[r, pl.ds(j * 16, 16)] = jnp.zeros((16,), jnp.float32)
      return carry
    lax.fori_loop(0, _ZR, zrow, 0)
    row0 = sid * _RPS
    for b in range(_RPS // _ZR):
      pltpu.sync_copy(z_v, acc.at[pl.ds(row0 + b * _ZR, _ZR)])
    plsc.subcore_barrier()

    e_base = wid * _EPW

    def chunk(c, carry):
      e0 = e_base + c * _CH
      pltpu.sync_copy(src.at[pl.ds(e0, _CH)], src_v)
      pltpu.sync_copy(dst.at[pl.ds(e0, _CH)], dst_v)
      pltpu.sync_copy(et.at[pl.ds(e0, _CH)], et_v)
      pltpu.async_copy(table.at[src_v], rows_v, sem).wait()

      def group(g, c2):
        tv = et_v[pl.ds(g * 16, 16)]
        for lane in range(16):
          t = tv[lane]
          e = g * 16 + lane
          for j in range(nj):
            sl = pl.ds(j * 16, 16)
            rows_v[e, sl] = rows_v[e, sl] * rel_v[t, sl]
        return c2
      lax.fori_loop(0, _CH // 16, group, 0)
      pltpu.sync_copy(rows_v, acc.at[dst_v], add=True)
      return carry
    lax.fori_loop(0, _NCHUNK, chunk, 0)

    plsc.subcore_barrier()
    pltpu.sync_copy(acc.at[pl.ds(row0, _RPS)], out.at[cid, pl.ds(row0, _RPS)])

  return pl.kernel(
      body,
      out_type=jax.ShapeDtypeStruct((_NC, _NP, dp), jnp.float32),
      mesh=mesh,
      scratch_types=[
          pltpu.VMEM_SHARED((_NP, dp), jnp.float32),
          pltpu.VMEM((_T, dp), jnp.float32),
          pltpu.VMEM((_CH, dp), jnp.float32),
          pltpu.VMEM((_CH,), jnp.int32),
          pltpu.VMEM((_CH,), jnp.int32),
          pltpu.VMEM((_CH,), jnp.int32),
          pltpu.VMEM((_ZR, dp), jnp.float32),
          pltpu.SemaphoreType.DMA,
      ],
  )


_edge_pass_1 = _make_edge_pass(_DP1)
_edge_pass_2 = _make_edge_pass(_DP2)


def _prep_body(x_ref, tn_ref, rb_ref, xp_ref, r1_ref):
  tn = tn_ref[0]
  rows = lax.broadcasted_iota(jnp.int32, (_N, 1), 0)
  xm = jnp.where(rows == tn, 0.0, x_ref[...])
  xp_ref[:, :_D] = xm
  xp_ref[:, _D:_D + 1] = jnp.ones((_N, 1), jnp.float32)
  xp_ref[:, _D + 1:] = jnp.zeros((_N, _DP1 - _D - 1), jnp.float32)
  rb = rb_ref[...]
  r1_ref[: _T // 2, :_D] = rb
  r1_ref[_T // 2:, :_D] = rb
  r1_ref[:, _D:_D + 1] = jnp.ones((_T, 1), jnp.float32)
  r1_ref[:, _D + 1:] = jnp.zeros((_T, _DP1 - _D - 1), jnp.float32)


def _dense1_body(p1_ref, xp_ref, w1m_ref, w1r_ref, b1_ref, r1_ref, rw_ref,
                 h_ref, r2_ref):
  p = p1_ref[0, :_N] + p1_ref[1, :_N]
  degc = jnp.maximum(p[:, _D:_D + 1], 1.0)
  feat = p[:, :_D] / degc
  h = (jnp.dot(feat, w1m_ref[...], preferred_element_type=jnp.float32)
       + jnp.dot(xp_ref[:, :_D], w1r_ref[...],
                 preferred_element_type=jnp.float32)
       + b1_ref[...])
  h_ref[:, :_H] = h
  h_ref[:, _H:_H + 1] = jnp.ones((_N, 1), jnp.float32)
  h_ref[:, _H + 1:] = jnp.zeros((_N, _DP2 - _H - 1), jnp.float32)
  r2 = jnp.dot(r1_ref[:, :_D], rw_ref[...], preferred_element_type=jnp.float32)
  r2_ref[:, :_H] = r2
  r2_ref[:, _H:_H + 1] = jnp.ones((_T, 1), jnp.float32)
  r2_ref[:, _H + 1:] = jnp.zeros((_T, _DP2 - _H - 1), jnp.float32)


def _dense2_body(p2_ref, hp_ref, w2m_ref, w2r_ref, b2_ref, o_ref):
  p = p2_ref[0, :_N] + p2_ref[1, :_N]
  degc = jnp.maximum(p[:, _H:_H + 1], 1.0)
  feat = p[:, :_H] / degc
  out = (jnp.dot(feat, w2m_ref[...], preferred_element_type=jnp.float32)
         + jnp.dot(hp_ref[:, :_H], w2r_ref[...],
                   preferred_element_type=jnp.float32)
         + b2_ref[...])
  m = jnp.max(out, axis=1, keepdims=True)
  s = out - m
  lse = jnp.log(jnp.sum(jnp.exp(s), axis=1, keepdims=True))
  o_ref[...] = s - lse


def kernel(x, rel_base, rel_weight, W1_msg, W1_root, b1, W2_msg, W2_root, b2,
           edge_index, edge_type, target_node):
  src = edge_index[0]
  dst = edge_index[1]
  tn = jnp.asarray(target_node, jnp.int32).reshape(1)

  xp, r1p = pl.pallas_call(
      _prep_body,
      out_shape=(
          jax.ShapeDtypeStruct((_N, _DP1), jnp.float32),
          jax.ShapeDtypeStruct((_T, _DP1), jnp.float32),
      ),
      in_specs=[
          pl.BlockSpec(memory_space=pltpu.VMEM),
          pl.BlockSpec(memory_space=pltpu.SMEM),
          pl.BlockSpec(memory_space=pltpu.VMEM),
      ],
  )(x, tn, rel_base)

  p1 = _edge_pass_1(xp, r1p, src, dst, edge_type)

  hp, r2p = pl.pallas_call(
      _dense1_body,
      out_shape=(
          jax.ShapeDtypeStruct((_N, _DP2), jnp.float32),
          jax.ShapeDtypeStruct((_T, _DP2), jnp.float32),
      ),
  )(p1, xp, W1_msg, W1_root, b1.reshape(1, _H), r1p, rel_weight)

  p2 = _edge_pass_2(hp, r2p, src, dst, edge_type)

  out = pl.pallas_call(
      _dense2_body,
      out_shape=jax.ShapeDtypeStruct((_N, _C), jnp.float32),
  )(p2, hp, W2_msg, W2_root, b2.reshape(1, _C))
  return out
